# baseline (speedup 1.0000x reference)
import jax
import jax.numpy as jnp
from jax import lax
from jax.experimental import pallas as pl
from jax.experimental.pallas import tpu as pltpu
from jax.experimental.pallas import tpu_sc as plsc

N_NODES_ = 10000
N_EDGES_ = 320000
HID_ = 64

NC = 2
NS = 16
NW = NC * NS

CHUNK = 128
KG = 5
KP = 4
EPT = N_EDGES_ // NW
NCH = 80
EPT_PAD = NCH * CHUNK
E_PAD = NW * EPT_PAD

NROW = 10112
RPT = NROW // NS

_mesh = plsc.VectorSubcoreMesh(
    core_axis_name="c", subcore_axis_name="s", num_cores=NC, num_subcores=NS)


def _make_seg_sum(with_counts):
  def body_fn(*args):
    if with_counts:
      (y_hbm, srcg_hbm, dstg_hbm, z64_hbm, z8_hbm, ones_hbm,
       sums_hbm, cnts_hbm,
       src_v, dst_v, rows_v, ones_v, accum, cacc, sems) = args
    else:
      (y_hbm, srcg_hbm, dstg_hbm, z64_hbm,
       sums_hbm,
       src_v, dst_v, rows_v, accum, sems) = args
    c = lax.axis_index("c")
    s = lax.axis_index("s")
    wid = s * NC + c
    pltpu.sync_copy(srcg_hbm.at[wid], src_v)
    pltpu.sync_copy(dstg_hbm.at[wid], dst_v)
    if with_counts:
      pltpu.sync_copy(ones_hbm, ones_v)
    pltpu.sync_copy(z64_hbm.at[pl.ds(s * RPT, RPT)],
                    accum.at[pl.ds(s * RPT, RPT)])
    if with_counts:
      pltpu.sync_copy(z8_hbm.at[pl.ds(s * RPT, RPT)],
                      cacc.at[pl.ds(s * RPT, RPT)])
    plsc.subcore_barrier()

    def body(j, carry):
      base = j * KG
      cps = [pltpu.async_copy(y_hbm.at[src_v.at[base + b]],
                              rows_v.at[b], sems.at[b])
             for b in range(KG)]
      for b in range(KG):
        cps[b].wait()
        pltpu.sync_copy(rows_v.at[b], accum.at[dst_v.at[base + b]], add=True)
        if with_counts:
          pltpu.sync_copy(ones_v, cacc.at[dst_v.at[base + b]], add=True)
      return carry

    lax.fori_loop(0, NCH // KG, body, 0)
    plsc.subcore_barrier()
    pltpu.sync_copy(accum.at[pl.ds(s * RPT, RPT)],
                    sums_hbm.at[c, pl.ds(s * RPT, RPT)])
    if with_counts:
      pltpu.sync_copy(cacc.at[pl.ds(s * RPT, RPT)],
                      cnts_hbm.at[c, pl.ds(s * RPT, RPT)])

  out_type = [jax.ShapeDtypeStruct((NC, NROW, HID_), jnp.float32)]
  scratch = [
      pltpu.VMEM((NCH, CHUNK), jnp.int32),
      pltpu.VMEM((NCH, CHUNK), jnp.int32),
      pltpu.VMEM((KG, CHUNK, HID_), jnp.float32),
  ]
  if with_counts:
    out_type.append(jax.ShapeDtypeStruct((NC, NROW, 8), jnp.float32))
    scratch.append(pltpu.VMEM((CHUNK, 8), jnp.float32))
  scratch.append(pltpu.VMEM_SHARED((NROW, HID_), jnp.float32))
  if with_counts:
    scratch.append(pltpu.VMEM_SHARED((NROW, 8), jnp.float32))
  scratch.append(pltpu.SemaphoreType.DMA((KG,)))
  return pl.kernel(
      body_fn,
      out_type=out_type,
      mesh=_mesh,
      scratch_types=scratch,
      compiler_params=pltpu.CompilerParams(use_tc_tiling_on_sc=False),
  )


_seg_sum_counts = _make_seg_sum(True)
_seg_sum_plain = _make_seg_sum(False)


def _pair_gather_body(a_hbm, b_hbm, srcg_hbm, dstg_hbm,
                      ga_hbm, gb_hbm,
                      src_v, dst_v, rows_a, rows_b, gsems, wsems):
  c = lax.axis_index("c")
  s = lax.axis_index("s")
  wid = s * NC + c
  pltpu.sync_copy(srcg_hbm.at[wid], src_v)
  pltpu.sync_copy(dstg_hbm.at[wid], dst_v)
  base_e = wid * EPT_PAD

  def _drain(b):
    pltpu.make_async_copy(rows_a.at[b], ga_hbm.at[pl.ds(base_e, CHUNK)],
                          wsems.at[b]).wait()
    pltpu.make_async_copy(rows_b.at[b], gb_hbm.at[pl.ds(base_e, CHUNK)],
                          wsems.at[KP + b]).wait()

  def group(j, carry):
    gbase = j * KP

    @pl.when(j > 0)
    def _():
      for b in range(KP):
        _drain(b)

    cps = []
    for b in range(KP):
      cps.append(pltpu.async_copy(a_hbm.at[src_v.at[gbase + b]],
                                  rows_a.at[b], gsems.at[b]))
      cps.append(pltpu.async_copy(b_hbm.at[dst_v.at[gbase + b]],
                                  rows_b.at[b], gsems.at[KP + b]))
    for b in range(KP):
      off = base_e + (gbase + b) * CHUNK
      cps[2 * b].wait()
      pltpu.async_copy(rows_a.at[b], ga_hbm.at[pl.ds(off, CHUNK)],
                       wsems.at[b])
      cps[2 * b + 1].wait()
      pltpu.async_copy(rows_b.at[b], gb_hbm.at[pl.ds(off, CHUNK)],
                       wsems.at[KP + b])
    return carry

  lax.fori_loop(0, NCH // KP, group, 0)
  for b in range(KP):
    _drain(b)


_pair_gather = pl.kernel(
    _pair_gather_body,
    out_type=[
        jax.ShapeDtypeStruct((E_PAD, HID_), jnp.float32),
        jax.ShapeDtypeStruct((E_PAD, HID_), jnp.float32),
    ],
    mesh=_mesh,
    scratch_types=[
        pltpu.VMEM((NCH, CHUNK), jnp.int32),
        pltpu.VMEM((NCH, CHUNK), jnp.int32),
        pltpu.VMEM((KP, CHUNK, HID_), jnp.float32),
        pltpu.VMEM((KP, CHUNK, HID_), jnp.float32),
        pltpu.SemaphoreType.DMA((2 * KP,)),
        pltpu.SemaphoreType.DMA((2 * KP,)),
    ],
    compiler_params=pltpu.CompilerParams(use_tc_tiling_on_sc=False),
)


def _tc_in_body(x_ref, wl_ref, wr_ref, y_ref, r_ref):
  x = x_ref[...]
  y_ref[...] = jnp.dot(x, wl_ref[...], preferred_element_type=jnp.float32)
  r_ref[...] = jnp.dot(x, wr_ref[...], preferred_element_type=jnp.float32)


def _tc_mid_body(sums_ref, cnts_ref, r_ref, bl_ref, wl_ref, wr_ref,
                 y_ref, r2_ref):
  s = sums_ref[0, :N_NODES_, :] + sums_ref[1, :N_NODES_, :]
  cnt = cnts_ref[0, :N_NODES_, 0:1] + cnts_ref[1, :N_NODES_, 0:1]
  inv = 1.0 / jnp.maximum(cnt, 1.0)
  h = jnp.maximum(s * inv + r_ref[...] + bl_ref[...], 0.0)
  y_ref[...] = jnp.dot(h, wl_ref[...], preferred_element_type=jnp.float32)
  r2_ref[...] = jnp.dot(h, wr_ref[...], preferred_element_type=jnp.float32)


def _tc_ab_body(sums_ref, cnts_ref, r_ref, bl_ref, wm1_ref, bm1_ref,
                a_ref, b_ref):
  s = sums_ref[0, :N_NODES_, :] + sums_ref[1, :N_NODES_, :]
  cnt = cnts_ref[0, :N_NODES_, 0:1] + cnts_ref[1, :N_NODES_, 0:1]
  inv = 1.0 / jnp.maximum(cnt, 1.0)
  h = jnp.maximum(s * inv + r_ref[...] + bl_ref[...], 0.0)
  a_ref[...] = (jnp.dot(h, wm1_ref[:HID_, :], preferred_element_type=jnp.float32)
                + bm1_ref[...])
  b_ref[...] = jnp.dot(h, wm1_ref[HID_:, :], preferred_element_type=jnp.float32)


EBLK = 1024


def _tc_edge_body(ga_ref, gb_ref, wm2_ref, bm2_ref, out_ref):
  h = jnp.maximum(ga_ref[...] + gb_ref[...], 0.0)
  o4 = (jnp.dot(h, wm2_ref[...], preferred_element_type=jnp.float32)
        + bm2_ref[...])
  out_ref[:EBLK, :] = o4[:, 0:2]
  out_ref[EBLK:, :] = o4[:, 2:4]


def kernel(x, edge_index, Wl1, Wr1, bl1, Wl2, Wr2, bl2, Wm1, bm1, Wm2, bm2):
  src = edge_index[0].astype(jnp.int32)
  dst = edge_index[1].astype(jnp.int32)
  pad = E_PAD - N_EDGES_
  pad_dst = N_NODES_ + (jnp.arange(pad, dtype=jnp.int32) % (NROW - N_NODES_))
  src_f = jnp.pad(src, (0, pad))
  dst_f = jnp.concatenate([dst, pad_dst])
  q = jnp.arange(E_PAD, dtype=jnp.int32)
  order = (q // (2 * EBLK)) * (2 * EBLK) + (q % 2) * EBLK + (q % (2 * EBLK)) // 2
  src_p = src_f[order].reshape(NW, NCH, CHUNK)
  dst_p = dst_f[order].reshape(NW, NCH, CHUNK)

  z64 = jnp.zeros((NROW, HID_), jnp.float32)
  z8 = jnp.zeros((NROW, 8), jnp.float32)
  ones8 = jnp.ones((CHUNK, 8), jnp.float32)

  y1, r1 = pl.pallas_call(
      _tc_in_body,
      out_shape=[jax.ShapeDtypeStruct((N_NODES_, HID_), jnp.float32)] * 2,
  )(x, Wl1, Wr1)

  sums1, cnts = _seg_sum_counts(y1, src_p, dst_p, z64, z8, ones8)

  y2, r2 = pl.pallas_call(
      _tc_mid_body,
      out_shape=[jax.ShapeDtypeStruct((N_NODES_, HID_), jnp.float32)] * 2,
  )(sums1, cnts, r1, bl1.reshape(1, HID_), Wl2, Wr2)

  (sums2,) = _seg_sum_plain(y2, src_p, dst_p, z64)

  A, B = pl.pallas_call(
      _tc_ab_body,
      out_shape=[jax.ShapeDtypeStruct((N_NODES_, HID_), jnp.float32)] * 2,
  )(sums2, cnts, r2, bl2.reshape(1, HID_), Wm1, bm1.reshape(1, HID_))

  ga, gb = _pair_gather(A, B, src_p, dst_p)

  ga2 = ga.reshape(E_PAD // 2, 2 * HID_)
  gb2 = gb.reshape(E_PAD // 2, 2 * HID_)
  w2x = jnp.zeros((2 * HID_, 4), jnp.float32)
  w2x = w2x.at[:HID_, :2].set(Wm2).at[HID_:, 2:].set(Wm2)
  bm2x = jnp.concatenate([bm2, bm2]).reshape(1, 4)

  n_eblk = (N_EDGES_ + 2 * EBLK - 1) // (2 * EBLK)
  out = pl.pallas_call(
      _tc_edge_body,
      grid=(n_eblk,),
      in_specs=[
          pl.BlockSpec((EBLK, 2 * HID_), lambda i: (i, 0)),
          pl.BlockSpec((EBLK, 2 * HID_), lambda i: (i, 0)),
          pl.BlockSpec((2 * HID_, 4), lambda i: (0, 0)),
          pl.BlockSpec((1, 4), lambda i: (0, 0)),
      ],
      out_specs=pl.BlockSpec((2 * EBLK, 2), lambda i: (i, 0)),
      out_shape=jax.ShapeDtypeStruct((N_EDGES_, 2), jnp.float32),
  )(ga2, gb2, w2x, bm2x)
  return out


# speedup vs baseline: 1.1093x; 1.1093x over previous
import jax
import jax.numpy as jnp
from jax import lax
from jax.experimental import pallas as pl
from jax.experimental.pallas import tpu as pltpu
from jax.experimental.pallas import tpu_sc as plsc

N_NODES_ = 10000
N_EDGES_ = 320000
HID_ = 64

NC = 2
NS = 16
NW = NC * NS

CHUNK = 128
KG = 5
KP = 4
EPT = N_EDGES_ // NW
NCH = 80
EPT_PAD = NCH * CHUNK
E_PAD = NW * EPT_PAD

NROW = 10112
RPT = NROW // NS

STEAL = 20
THIEF_C = 0
CH_V = NCH - STEAL

_mesh = plsc.VectorSubcoreMesh(
    core_axis_name="c", subcore_axis_name="s", num_cores=NC, num_subcores=NS)


def _make_seg_sum(with_counts):
  def body_fn(*args):
    if with_counts:
      (y_hbm, srcg_hbm, dstg_hbm, z64_hbm, z8_hbm, ones_hbm,
       sums_hbm, cnts_hbm,
       src_v, dst_v, src_x, dst_x, rows_v, ones_v, accum, cacc, sems) = args
    else:
      (y_hbm, srcg_hbm, dstg_hbm, z64_hbm,
       sums_hbm,
       src_v, dst_v, src_x, dst_x, rows_v, accum, sems) = args
    c = lax.axis_index("c")
    s = lax.axis_index("s")
    wid = s * NC + c
    vict = s * NC + (1 - THIEF_C)
    pltpu.sync_copy(srcg_hbm.at[wid], src_v)
    pltpu.sync_copy(dstg_hbm.at[wid], dst_v)

    @pl.when(c == THIEF_C)
    def _():
      pltpu.sync_copy(srcg_hbm.at[vict, pl.ds(CH_V, STEAL)], src_x)
      pltpu.sync_copy(dstg_hbm.at[vict, pl.ds(CH_V, STEAL)], dst_x)
    if with_counts:
      pltpu.sync_copy(ones_hbm, ones_v)
    pltpu.sync_copy(z64_hbm.at[pl.ds(s * RPT, RPT)],
                    accum.at[pl.ds(s * RPT, RPT)])
    if with_counts:
      pltpu.sync_copy(z8_hbm.at[pl.ds(s * RPT, RPT)],
                      cacc.at[pl.ds(s * RPT, RPT)])
    plsc.subcore_barrier()

    def make_body(sref, dref):
      def body(j, carry):
        base = j * KG
        cps = [pltpu.async_copy(y_hbm.at[sref.at[base + b]],
                                rows_v.at[b], sems.at[b])
               for b in range(KG)]
        for b in range(KG):
          cps[b].wait()
          pltpu.sync_copy(rows_v.at[b], accum.at[dref.at[base + b]], add=True)
          if with_counts:
            pltpu.sync_copy(ones_v, cacc.at[dref.at[base + b]], add=True)
        return carry
      return body

    ngroups = jnp.where(c == THIEF_C, NCH // KG, CH_V // KG)
    lax.fori_loop(0, ngroups, make_body(src_v, dst_v), 0)

    @pl.when(c == THIEF_C)
    def _():
      lax.fori_loop(0, STEAL // KG, make_body(src_x, dst_x), 0)

    plsc.subcore_barrier()
    pltpu.sync_copy(accum.at[pl.ds(s * RPT, RPT)],
                    sums_hbm.at[c, pl.ds(s * RPT, RPT)])
    if with_counts:
      pltpu.sync_copy(cacc.at[pl.ds(s * RPT, RPT)],
                      cnts_hbm.at[c, pl.ds(s * RPT, RPT)])

  out_type = [jax.ShapeDtypeStruct((NC, NROW, HID_), jnp.float32)]
  scratch = [
      pltpu.VMEM((NCH, CHUNK), jnp.int32),
      pltpu.VMEM((NCH, CHUNK), jnp.int32),
      pltpu.VMEM((STEAL, CHUNK), jnp.int32),
      pltpu.VMEM((STEAL, CHUNK), jnp.int32),
      pltpu.VMEM((KG, CHUNK, HID_), jnp.float32),
  ]
  if with_counts:
    out_type.append(jax.ShapeDtypeStruct((NC, NROW, 8), jnp.float32))
    scratch.append(pltpu.VMEM((CHUNK, 8), jnp.float32))
  scratch.append(pltpu.VMEM_SHARED((NROW, HID_), jnp.float32))
  if with_counts:
    scratch.append(pltpu.VMEM_SHARED((NROW, 8), jnp.float32))
  scratch.append(pltpu.SemaphoreType.DMA((KG,)))
  return pl.kernel(
      body_fn,
      out_type=out_type,
      mesh=_mesh,
      scratch_types=scratch,
      compiler_params=pltpu.CompilerParams(use_tc_tiling_on_sc=False),
  )


_seg_sum_counts = _make_seg_sum(True)
_seg_sum_plain = _make_seg_sum(False)


def _pair_gather_body(a_hbm, b_hbm, srcg_hbm, dstg_hbm,
                      ga_hbm, gb_hbm,
                      src_v, dst_v, src_x, dst_x, rows_a, rows_b,
                      gsems, wsems):
  c = lax.axis_index("c")
  s = lax.axis_index("s")
  wid = s * NC + c
  vict = s * NC + (1 - THIEF_C)
  pltpu.sync_copy(srcg_hbm.at[wid], src_v)
  pltpu.sync_copy(dstg_hbm.at[wid], dst_v)

  @pl.when(c == THIEF_C)
  def _():
    pltpu.sync_copy(srcg_hbm.at[vict, pl.ds(CH_V, STEAL)], src_x)
    pltpu.sync_copy(dstg_hbm.at[vict, pl.ds(CH_V, STEAL)], dst_x)

  def _drain(b):
    pltpu.make_async_copy(rows_a.at[b], ga_hbm.at[pl.ds(0, CHUNK)],
                          wsems.at[b]).wait()
    pltpu.make_async_copy(rows_b.at[b], gb_hbm.at[pl.ds(0, CHUNK)],
                          wsems.at[KP + b]).wait()

  def make_group(sref, dref, base_e, always_drain):
    def group(j, carry):
      gbase = j * KP

      @pl.when(always_drain | (j > 0))
      def _():
        for b in range(KP):
          _drain(b)

      cps = []
      for b in range(KP):
        cps.append(pltpu.async_copy(a_hbm.at[sref.at[gbase + b]],
                                    rows_a.at[b], gsems.at[b]))
        cps.append(pltpu.async_copy(b_hbm.at[dref.at[gbase + b]],
                                    rows_b.at[b], gsems.at[KP + b]))
      for b in range(KP):
        off = base_e + (gbase + b) * CHUNK
        cps[2 * b].wait()
        pltpu.async_copy(rows_a.at[b], ga_hbm.at[pl.ds(off, CHUNK)],
                         wsems.at[b])
        cps[2 * b + 1].wait()
        pltpu.async_copy(rows_b.at[b], gb_hbm.at[pl.ds(off, CHUNK)],
                         wsems.at[KP + b])
      return carry
    return group

  ngroups = jnp.where(c == THIEF_C, NCH // KP, CH_V // KP)
  lax.fori_loop(0, ngroups,
                make_group(src_v, dst_v, wid * EPT_PAD, False), 0)

  @pl.when(c == THIEF_C)
  def _():
    lax.fori_loop(0, STEAL // KP,
                  make_group(src_x, dst_x,
                             vict * EPT_PAD + CH_V * CHUNK, True), 0)

  for b in range(KP):
    _drain(b)


_pair_gather = pl.kernel(
    _pair_gather_body,
    out_type=[
        jax.ShapeDtypeStruct((E_PAD, HID_), jnp.float32),
        jax.ShapeDtypeStruct((E_PAD, HID_), jnp.float32),
    ],
    mesh=_mesh,
    scratch_types=[
        pltpu.VMEM((NCH, CHUNK), jnp.int32),
        pltpu.VMEM((NCH, CHUNK), jnp.int32),
        pltpu.VMEM((STEAL, CHUNK), jnp.int32),
        pltpu.VMEM((STEAL, CHUNK), jnp.int32),
        pltpu.VMEM((KP, CHUNK, HID_), jnp.float32),
        pltpu.VMEM((KP, CHUNK, HID_), jnp.float32),
        pltpu.SemaphoreType.DMA((2 * KP,)),
        pltpu.SemaphoreType.DMA((2 * KP,)),
    ],
    compiler_params=pltpu.CompilerParams(use_tc_tiling_on_sc=False),
)


def _tc_in_body(x_ref, wl_ref, wr_ref, y_ref, r_ref):
  x = x_ref[...]
  y_ref[...] = jnp.dot(x, wl_ref[...], preferred_element_type=jnp.float32)
  r_ref[...] = jnp.dot(x, wr_ref[...], preferred_element_type=jnp.float32)


def _tc_mid_body(sums_ref, cnts_ref, r_ref, bl_ref, wl_ref, wr_ref,
                 y_ref, r2_ref):
  s = sums_ref[0, :N_NODES_, :] + sums_ref[1, :N_NODES_, :]
  cnt = cnts_ref[0, :N_NODES_, 0:1] + cnts_ref[1, :N_NODES_, 0:1]
  inv = 1.0 / jnp.maximum(cnt, 1.0)
  h = jnp.maximum(s * inv + r_ref[...] + bl_ref[...], 0.0)
  y_ref[...] = jnp.dot(h, wl_ref[...], preferred_element_type=jnp.float32)
  r2_ref[...] = jnp.dot(h, wr_ref[...], preferred_element_type=jnp.float32)


def _tc_ab_body(sums_ref, cnts_ref, r_ref, bl_ref, wm1_ref, bm1_ref,
                a_ref, b_ref):
  s = sums_ref[0, :N_NODES_, :] + sums_ref[1, :N_NODES_, :]
  cnt = cnts_ref[0, :N_NODES_, 0:1] + cnts_ref[1, :N_NODES_, 0:1]
  inv = 1.0 / jnp.maximum(cnt, 1.0)
  h = jnp.maximum(s * inv + r_ref[...] + bl_ref[...], 0.0)
  a_ref[...] = (jnp.dot(h, wm1_ref[:HID_, :], preferred_element_type=jnp.float32)
                + bm1_ref[...])
  b_ref[...] = jnp.dot(h, wm1_ref[HID_:, :], preferred_element_type=jnp.float32)


EBLK = 1024


def _tc_edge_body(ga_ref, gb_ref, wm2_ref, bm2_ref, out_ref):
  h = jnp.maximum(ga_ref[...] + gb_ref[...], 0.0)
  o4 = (jnp.dot(h, wm2_ref[...], preferred_element_type=jnp.float32)
        + bm2_ref[...])
  out_ref[:EBLK, :] = o4[:, 0:2]
  out_ref[EBLK:, :] = o4[:, 2:4]


def kernel(x, edge_index, Wl1, Wr1, bl1, Wl2, Wr2, bl2, Wm1, bm1, Wm2, bm2):
  src = edge_index[0].astype(jnp.int32)
  dst = edge_index[1].astype(jnp.int32)
  pad = E_PAD - N_EDGES_
  pad_dst = N_NODES_ + (jnp.arange(pad, dtype=jnp.int32) % (NROW - N_NODES_))
  src_f = jnp.pad(src, (0, pad))
  dst_f = jnp.concatenate([dst, pad_dst])
  q = jnp.arange(E_PAD, dtype=jnp.int32)
  order = (q // (2 * EBLK)) * (2 * EBLK) + (q % 2) * EBLK + (q % (2 * EBLK)) // 2
  src_p = src_f[order].reshape(NW, NCH, CHUNK)
  dst_p = dst_f[order].reshape(NW, NCH, CHUNK)

  z64 = jnp.zeros((NROW, HID_), jnp.float32)
  z8 = jnp.zeros((NROW, 8), jnp.float32)
  ones8 = jnp.ones((CHUNK, 8), jnp.float32)

  y1, r1 = pl.pallas_call(
      _tc_in_body,
      out_shape=[jax.ShapeDtypeStruct((N_NODES_, HID_), jnp.float32)] * 2,
  )(x, Wl1, Wr1)

  sums1, cnts = _seg_sum_counts(y1, src_p, dst_p, z64, z8, ones8)

  y2, r2 = pl.pallas_call(
      _tc_mid_body,
      out_shape=[jax.ShapeDtypeStruct((N_NODES_, HID_), jnp.float32)] * 2,
  )(sums1, cnts, r1, bl1.reshape(1, HID_), Wl2, Wr2)

  (sums2,) = _seg_sum_plain(y2, src_p, dst_p, z64)

  A, B = pl.pallas_call(
      _tc_ab_body,
      out_shape=[jax.ShapeDtypeStruct((N_NODES_, HID_), jnp.float32)] * 2,
  )(sums2, cnts, r2, bl2.reshape(1, HID_), Wm1, bm1.reshape(1, HID_))

  ga, gb = _pair_gather(A, B, src_p, dst_p)

  ga2 = ga.reshape(E_PAD // 2, 2 * HID_)
  gb2 = gb.reshape(E_PAD // 2, 2 * HID_)
  w2x = jnp.zeros((2 * HID_, 4), jnp.float32)
  w2x = w2x.at[:HID_, :2].set(Wm2).at[HID_:, 2:].set(Wm2)
  bm2x = jnp.concatenate([bm2, bm2]).reshape(1, 4)

  n_eblk = (N_EDGES_ + 2 * EBLK - 1) // (2 * EBLK)
  out = pl.pallas_call(
      _tc_edge_body,
      grid=(n_eblk,),
      in_specs=[
          pl.BlockSpec((EBLK, 2 * HID_), lambda i: (i, 0)),
          pl.BlockSpec((EBLK, 2 * HID_), lambda i: (i, 0)),
          pl.BlockSpec((2 * HID_, 4), lambda i: (0, 0)),
          pl.BlockSpec((1, 4), lambda i: (0, 0)),
      ],
      out_specs=pl.BlockSpec((2 * EBLK, 2), lambda i: (i, 0)),
      out_shape=jax.ShapeDtypeStruct((N_EDGES_, 2), jnp.float32),
  )(ga2, gb2, w2x, bm2x)
  return out
